# Initial kernel scaffold; baseline (speedup 1.0000x reference)
import functools

import jax
import jax.numpy as jnp
from jax import lax
from jax.experimental import pallas as pl
from jax.experimental.pallas import tpu as pltpu
from jax.experimental.pallas import tpu_sc as plsc

_NC = 2
_NS = 16
_CH = 128
_DEGW = 16


def _make_edge_agg(n_table, d, n_acc, e_pad, with_deg):
  steps = e_pad // (_NC * _NS * _CH)
  rows_per_tile = n_acc // _NS
  zsteps = rows_per_tile // _CH
  dpg = d // 16

  mesh = plsc.VectorSubcoreMesh(core_axis_name="c", subcore_axis_name="s",
                                num_cores=_NC, num_subcores=_NS)
  out_type = [jax.ShapeDtypeStruct((_NC, n_acc, d), jnp.float32)]
  scratch = [
      pltpu.VMEM((steps, _CH), jnp.int32),
      pltpu.VMEM((steps, _CH), jnp.int32),
      pltpu.VMEM((_CH, d), jnp.float32),
      pltpu.VMEM_SHARED((n_acc, d), jnp.float32),
      pltpu.SemaphoreType.DMA,
  ]
  if with_deg:
    out_type.append(jax.ShapeDtypeStruct((_NC, n_acc, _DEGW), jnp.float32))
    scratch = scratch[:3] + [
        pltpu.VMEM((_CH, _DEGW), jnp.float32),
        pltpu.VMEM_SHARED((n_acc, _DEGW), jnp.float32),
    ] + scratch[3:]

  def body(*refs):
    if with_deg:
      (table, srci, dsti, out, degout,
       srcv, dstv, gbuf, onesv, degacc, acc, sem) = refs
    else:
      (table, srci, dsti, out,
       srcv, dstv, gbuf, acc, sem) = refs

    c = lax.axis_index("c")
    s = lax.axis_index("s")
    wid = c * _NS + s
    row0 = s * rows_per_tile

    def zstore(i, _):
      gbuf[i // dpg, pl.ds((i % dpg) * 16, 16)] = jnp.zeros((16,), jnp.float32)
      return 0
    lax.fori_loop(0, _CH * dpg, zstore, 0)

    def zcopy(k, _):
      pltpu.sync_copy(gbuf, acc.at[pl.ds(row0 + k * _CH, _CH)])
      return 0
    lax.fori_loop(0, zsteps, zcopy, 0)

    if with_deg:
      def zstore16(i, _):
        onesv[i, pl.ds(0, 16)] = jnp.zeros((16,), jnp.float32)
        return 0
      lax.fori_loop(0, _CH, zstore16, 0)
      def zcopy16(k, _):
        pltpu.sync_copy(onesv, degacc.at[pl.ds(row0 + k * _CH, _CH)])
        return 0
      lax.fori_loop(0, zsteps, zcopy16, 0)
      def ostore16(i, _):
        onesv[i, pl.ds(0, 16)] = jnp.ones((16,), jnp.float32)
        return 0
      lax.fori_loop(0, _CH, ostore16, 0)

    plsc.subcore_barrier()

    base = wid * steps
    pltpu.sync_copy(srci.at[pl.ds(base, steps)], srcv)
    pltpu.sync_copy(dsti.at[pl.ds(base, steps)], dstv)

    def step(j, _):
      pltpu.async_copy(table.at[srcv.at[j]], gbuf, sem).wait()
      pltpu.sync_copy(gbuf, acc.at[dstv.at[j]], add=True)
      if with_deg:
        pltpu.sync_copy(onesv, degacc.at[dstv.at[j]], add=True)
      return 0
    lax.fori_loop(0, steps, step, 0)

    plsc.subcore_barrier()

    def ocopy(k, _):
      r = row0 + k * _CH
      pltpu.sync_copy(acc.at[pl.ds(r, _CH)], out.at[c, pl.ds(r, _CH)])
      return 0
    lax.fori_loop(0, zsteps, ocopy, 0)
    if with_deg:
      pltpu.sync_copy(degacc.at[pl.ds(row0, rows_per_tile)],
                      degout.at[c, pl.ds(row0, rows_per_tile)])

  return pl.kernel(body, out_type=tuple(out_type), mesh=mesh,
                   scratch_types=tuple(scratch))


def _tc_layer1(p, degp, x, wl1t, bl1, wr1t, wl2t, wr2t, n, blk):
  d = x.shape[1]
  h2 = wl2t.shape[1]
  grid = (n // blk,)

  def body(p_ref, deg_ref, x_ref, wl1_ref, bl1_ref, wr1_ref, wl2_ref, wr2_ref,
           hl_ref, hr_ref):
    pv = p_ref[...]
    agg = pv[0] + pv[1]
    dg = deg_ref[0, :, 0:1] + deg_ref[1, :, 0:1]
    rdeg = 1.0 / jnp.maximum(dg, 1.0)
    h = (jnp.dot(agg * rdeg, wl1_ref[...], preferred_element_type=jnp.float32)
         + bl1_ref[...]
         + jnp.dot(x_ref[...], wr1_ref[...], preferred_element_type=jnp.float32))
    h = jnp.maximum(h, 0.0)
    hl_ref[...] = jnp.dot(h, wl2_ref[...], preferred_element_type=jnp.float32)
    hr_ref[...] = jnp.dot(h, wr2_ref[...], preferred_element_type=jnp.float32)

  return pl.pallas_call(
      body,
      grid=grid,
      in_specs=[
          pl.BlockSpec((_NC, blk, d), lambda i: (0, i, 0)),
          pl.BlockSpec((_NC, blk, _DEGW), lambda i: (0, i, 0)),
          pl.BlockSpec((blk, d), lambda i: (i, 0)),
          pl.BlockSpec((d, d), lambda i: (0, 0)),
          pl.BlockSpec((1, d), lambda i: (0, 0)),
          pl.BlockSpec((d, d), lambda i: (0, 0)),
          pl.BlockSpec((d, h2), lambda i: (0, 0)),
          pl.BlockSpec((d, h2), lambda i: (0, 0)),
      ],
      out_specs=[
          pl.BlockSpec((blk, h2), lambda i: (i, 0)),
          pl.BlockSpec((blk, h2), lambda i: (i, 0)),
      ],
      out_shape=[
          jax.ShapeDtypeStruct((n, h2), jnp.float32),
          jax.ShapeDtypeStruct((n, h2), jnp.float32),
      ],
  )(p, degp, x, wl1t, bl1, wr1t, wl2t, wr2t)


def _tc_layer2(q, degp, hr, bl2, n, blk):
  c = hr.shape[1]
  grid = (n // blk,)

  def body(q_ref, deg_ref, hr_ref, bl2_ref, o_ref):
    qv = q_ref[...]
    agg = qv[0] + qv[1]
    dg = deg_ref[0, :, 0:1] + deg_ref[1, :, 0:1]
    rdeg = 1.0 / jnp.maximum(dg, 1.0)
    o = agg * rdeg + bl2_ref[...] + hr_ref[...]
    o_ref[...] = jax.nn.sigmoid(o)

  return pl.pallas_call(
      body,
      grid=grid,
      in_specs=[
          pl.BlockSpec((_NC, blk, c), lambda i: (0, i, 0)),
          pl.BlockSpec((_NC, blk, _DEGW), lambda i: (0, i, 0)),
          pl.BlockSpec((blk, c), lambda i: (i, 0)),
          pl.BlockSpec((1, c), lambda i: (0, 0)),
      ],
      out_specs=pl.BlockSpec((blk, c), lambda i: (i, 0)),
      out_shape=jax.ShapeDtypeStruct((n, c), jnp.float32),
  )(q, degp, hr, bl2)


def kernel(x, edge_index, Wl1, bl1, Wr1, Wl2, bl2, Wr2):
  n, d = x.shape
  h2 = Wl2.shape[0]
  e = edge_index.shape[1]

  egrain = _NC * _NS * _CH * 2
  e_pad = -(-e // egrain) * egrain
  n_acc = -(-(n + 1) // (_NS * _CH)) * (_NS * _CH)

  src = edge_index[0]
  dst = edge_index[1]
  pad = e_pad - e
  src_p = jnp.concatenate([src, jnp.zeros((pad,), jnp.int32)]).reshape(
      e_pad // _CH, _CH)
  dst_p = jnp.concatenate([dst, jnp.full((pad,), n, jnp.int32)]).reshape(
      e_pad // _CH, _CH)

  agg1 = _make_edge_agg(n, d, n_acc, e_pad, with_deg=True)
  p, degp = agg1(x, src_p, dst_p)

  blk = 1000 if n % 1000 == 0 else 8 * (n // 8)
  hl, hr = _tc_layer1(p, degp, x, Wl1.T, bl1.reshape(1, -1), Wr1.T,
                      Wl2.T, Wr2.T, n, blk)

  agg2 = _make_edge_agg(n, h2, n_acc, e_pad, with_deg=False)
  (q,) = agg2(hl, src_p, dst_p)

  return _tc_layer2(q, degp, hr, bl2.reshape(1, -1), n, blk)


# speedup vs baseline: 3.8786x; 3.8786x over previous
import functools

import jax
import jax.numpy as jnp
from jax import lax
from jax.experimental import pallas as pl
from jax.experimental.pallas import tpu as pltpu
from jax.experimental.pallas import tpu_sc as plsc

_NC = 2
_NS = 16
_CH = 128
_DEGW = 16


def _make_edge_agg(d, n_acc, e_pad, with_deg):
  steps = e_pad // (_NC * _NS * _CH)
  rows_per_tile = n_acc // _NS
  zsteps = rows_per_tile // _CH
  dpg = d // 16

  mesh = plsc.VectorSubcoreMesh(core_axis_name="c", subcore_axis_name="s",
                                num_cores=_NC, num_subcores=_NS)
  out_type = [jax.ShapeDtypeStruct((_NC, n_acc, d), jnp.float32)]
  scratch = [
      pltpu.VMEM((steps, _CH), jnp.int32),
      pltpu.VMEM((steps, _CH), jnp.int32),
      pltpu.VMEM((_CH, d), jnp.float32),
      pltpu.VMEM_SHARED((n_acc, d), jnp.float32),
      pltpu.SemaphoreType.DMA,
  ]
  if with_deg:
    out_type.append(jax.ShapeDtypeStruct((_NC, n_acc, _DEGW), jnp.float32))
    scratch = scratch[:3] + [
        pltpu.VMEM((_CH, _DEGW), jnp.float32),
        pltpu.VMEM_SHARED((n_acc, _DEGW), jnp.float32),
    ] + scratch[3:]

  def body(*refs):
    if with_deg:
      (table, srci, dsti, out, degout,
       srcv, dstv, gbuf, onesv, degacc, acc, sem) = refs
    else:
      (table, srci, dsti, out,
       srcv, dstv, gbuf, acc, sem) = refs

    c = lax.axis_index("c")
    s = lax.axis_index("s")
    wid = c * _NS + s
    row0 = s * rows_per_tile

    def zstore(i, _):
      gbuf[i // dpg, pl.ds((i % dpg) * 16, 16)] = jnp.zeros((16,), jnp.float32)
      return 0
    lax.fori_loop(0, _CH * dpg, zstore, 0)

    def zcopy(k, _):
      pltpu.sync_copy(gbuf, acc.at[pl.ds(row0 + k * _CH, _CH)])
      return 0
    lax.fori_loop(0, zsteps, zcopy, 0)

    if with_deg:
      def zstore16(i, _):
        onesv[i, pl.ds(0, 16)] = jnp.zeros((16,), jnp.float32)
        return 0
      lax.fori_loop(0, _CH, zstore16, 0)
      def zcopy16(k, _):
        pltpu.sync_copy(onesv, degacc.at[pl.ds(row0 + k * _CH, _CH)])
        return 0
      lax.fori_loop(0, zsteps, zcopy16, 0)
      def ostore16(i, _):
        onesv[i, pl.ds(0, 16)] = jnp.ones((16,), jnp.float32)
        return 0
      lax.fori_loop(0, _CH, ostore16, 0)

    plsc.subcore_barrier()

    base = wid * steps
    pltpu.sync_copy(srci.at[pl.ds(base, steps)], srcv)
    pltpu.sync_copy(dsti.at[pl.ds(base, steps)], dstv)

    def step(j, _):
      pltpu.async_copy(table.at[srcv.at[j]], gbuf, sem).wait()
      pltpu.sync_copy(gbuf, acc.at[dstv.at[j]], add=True)
      if with_deg:
        pltpu.sync_copy(onesv, degacc.at[dstv.at[j]], add=True)
      return 0
    lax.fori_loop(0, steps, step, 0)

    plsc.subcore_barrier()

    def ocopy(k, _):
      r = row0 + k * _CH
      pltpu.sync_copy(acc.at[pl.ds(r, _CH)], out.at[c, pl.ds(r, _CH)])
      return 0
    lax.fori_loop(0, zsteps, ocopy, 0)
    if with_deg:
      pltpu.sync_copy(degacc.at[pl.ds(row0, rows_per_tile)],
                      degout.at[c, pl.ds(row0, rows_per_tile)])

  return pl.kernel(body, out_type=tuple(out_type), mesh=mesh,
                   scratch_types=tuple(scratch),
                   compiler_params=pltpu.CompilerParams(
                       use_tc_tiling_on_sc=False))


def _tc_layer1(pa, pb, degp, x, wl1t_a, wl1t_b, bl1, wr1t, wl2t, wr2t, n, blk):
  d = x.shape[1]
  dh = d // 2
  h2 = wl2t.shape[1]
  grid = (n // blk,)

  def body(pa_ref, pb_ref, deg_ref, x_ref, wl1a_ref, wl1b_ref, bl1_ref,
           wr1_ref, wl2_ref, wr2_ref, hl_ref, hr_ref):
    agg_a = pa_ref[0] + pa_ref[1]
    agg_b = pb_ref[0] + pb_ref[1]
    dg = deg_ref[0, :, 0:1] + deg_ref[1, :, 0:1]
    rdeg = 1.0 / jnp.maximum(dg, 1.0)
    h = (jnp.dot(agg_a * rdeg, wl1a_ref[...], preferred_element_type=jnp.float32)
         + jnp.dot(agg_b * rdeg, wl1b_ref[...], preferred_element_type=jnp.float32)
         + bl1_ref[...]
         + jnp.dot(x_ref[...], wr1_ref[...], preferred_element_type=jnp.float32))
    h = jnp.maximum(h, 0.0)
    hl_ref[...] = jnp.dot(h, wl2_ref[...], preferred_element_type=jnp.float32)
    hr_ref[...] = jnp.dot(h, wr2_ref[...], preferred_element_type=jnp.float32)

  return pl.pallas_call(
      body,
      grid=grid,
      in_specs=[
          pl.BlockSpec((_NC, blk, dh), lambda i: (0, i, 0)),
          pl.BlockSpec((_NC, blk, dh), lambda i: (0, i, 0)),
          pl.BlockSpec((_NC, blk, _DEGW), lambda i: (0, i, 0)),
          pl.BlockSpec((blk, d), lambda i: (i, 0)),
          pl.BlockSpec((dh, d), lambda i: (0, 0)),
          pl.BlockSpec((dh, d), lambda i: (0, 0)),
          pl.BlockSpec((1, d), lambda i: (0, 0)),
          pl.BlockSpec((d, d), lambda i: (0, 0)),
          pl.BlockSpec((d, h2), lambda i: (0, 0)),
          pl.BlockSpec((d, h2), lambda i: (0, 0)),
      ],
      out_specs=[
          pl.BlockSpec((blk, h2), lambda i: (i, 0)),
          pl.BlockSpec((blk, h2), lambda i: (i, 0)),
      ],
      out_shape=[
          jax.ShapeDtypeStruct((n, h2), jnp.float32),
          jax.ShapeDtypeStruct((n, h2), jnp.float32),
      ],
  )(pa, pb, degp, x, wl1t_a, wl1t_b, bl1, wr1t, wl2t, wr2t)


def _tc_layer2(q, degp, hr, bl2, n, blk):
  c = hr.shape[1]
  grid = (n // blk,)

  def body(q_ref, deg_ref, hr_ref, bl2_ref, o_ref):
    agg = q_ref[0] + q_ref[1]
    dg = deg_ref[0, :, 0:1] + deg_ref[1, :, 0:1]
    rdeg = 1.0 / jnp.maximum(dg, 1.0)
    o = agg * rdeg + bl2_ref[...] + hr_ref[...]
    o_ref[...] = jax.nn.sigmoid(o)

  return pl.pallas_call(
      body,
      grid=grid,
      in_specs=[
          pl.BlockSpec((_NC, blk, c), lambda i: (0, i, 0)),
          pl.BlockSpec((_NC, blk, _DEGW), lambda i: (0, i, 0)),
          pl.BlockSpec((blk, c), lambda i: (i, 0)),
          pl.BlockSpec((1, c), lambda i: (0, 0)),
      ],
      out_specs=pl.BlockSpec((blk, c), lambda i: (i, 0)),
      out_shape=jax.ShapeDtypeStruct((n, c), jnp.float32),
  )(q, degp, hr, bl2)


def kernel(x, edge_index, Wl1, bl1, Wr1, Wl2, bl2, Wr2):
  n, d = x.shape
  dh = d // 2
  h2 = Wl2.shape[0]
  e = edge_index.shape[1]

  egrain = _NC * _NS * _CH * 2
  e_pad = -(-e // egrain) * egrain
  n_acc = -(-(n + 1) // (_NS * _CH)) * (_NS * _CH)

  src = edge_index[0]
  dst = edge_index[1]
  pad = e_pad - e
  src_p = jnp.concatenate([src, jnp.zeros((pad,), jnp.int32)]).reshape(
      e_pad // _CH, _CH)
  dst_p = jnp.concatenate([dst, jnp.full((pad,), n, jnp.int32)]).reshape(
      e_pad // _CH, _CH)

  agg_deg = _make_edge_agg(dh, n_acc, e_pad, with_deg=True)
  agg = _make_edge_agg(dh, n_acc, e_pad, with_deg=False)

  xa = x[:, :dh]
  xb = x[:, dh:]
  pa, degp = agg_deg(xa, src_p, dst_p)
  (pb,) = agg(xb, src_p, dst_p)

  blk = 1000 if n % 1000 == 0 else 8 * (n // 8)
  hl, hr = _tc_layer1(pa, pb, degp, x, Wl1.T[:dh], Wl1.T[dh:],
                      bl1.reshape(1, -1), Wr1.T, Wl2.T, Wr2.T, n, blk)

  (q,) = agg(hl, src_p, dst_p)

  return _tc_layer2(q, degp, hr, bl2.reshape(1, -1), n, blk)


# speedup vs baseline: 4.5225x; 1.1660x over previous
import functools

import jax
import jax.numpy as jnp
from jax import lax
from jax.experimental import pallas as pl
from jax.experimental.pallas import tpu as pltpu
from jax.experimental.pallas import tpu_sc as plsc

_NC = 2
_NS = 16
_CH = 128
_DEGW = 16


def _make_edge_agg(d, n_acc, e_pad, with_deg):
  steps = e_pad // (_NC * _NS * _CH)
  rows_per_tile = n_acc // _NS
  zsteps = rows_per_tile // _CH
  dpg = d // 16

  mesh = plsc.VectorSubcoreMesh(core_axis_name="c", subcore_axis_name="s",
                                num_cores=_NC, num_subcores=_NS)
  out_type = [jax.ShapeDtypeStruct((_NC, n_acc, d), jnp.float32)]
  scratch = [
      pltpu.VMEM((steps, _CH), jnp.int32),
      pltpu.VMEM((steps, _CH), jnp.int32),
      pltpu.VMEM((_CH, d), jnp.float32),
      pltpu.VMEM((_CH, d), jnp.float32),
      pltpu.VMEM_SHARED((n_acc, d), jnp.float32),
      pltpu.SemaphoreType.DMA,
      pltpu.SemaphoreType.DMA,
  ]
  if with_deg:
    out_type.append(jax.ShapeDtypeStruct((_NC, n_acc, _DEGW), jnp.float32))
    scratch = scratch[:4] + [
        pltpu.VMEM((_CH, _DEGW), jnp.float32),
        pltpu.VMEM_SHARED((n_acc, _DEGW), jnp.float32),
    ] + scratch[4:]

  def body(*refs):
    if with_deg:
      (table, srci, dsti, out, degout,
       srcv, dstv, gbuf, gbuf1, onesv, degacc, acc, sem, sem1) = refs
    else:
      (table, srci, dsti, out,
       srcv, dstv, gbuf, gbuf1, acc, sem, sem1) = refs

    c = lax.axis_index("c")
    s = lax.axis_index("s")
    wid = c * _NS + s
    row0 = s * rows_per_tile

    def zstore(i, _):
      gbuf[i // dpg, pl.ds((i % dpg) * 16, 16)] = jnp.zeros((16,), jnp.float32)
      return 0
    lax.fori_loop(0, _CH * dpg, zstore, 0)

    def zcopy(k, _):
      pltpu.sync_copy(gbuf, acc.at[pl.ds(row0 + k * _CH, _CH)])
      return 0
    lax.fori_loop(0, zsteps, zcopy, 0)

    if with_deg:
      def zstore16(i, _):
        onesv[i, pl.ds(0, 16)] = jnp.zeros((16,), jnp.float32)
        return 0
      lax.fori_loop(0, _CH, zstore16, 0)
      def zcopy16(k, _):
        pltpu.sync_copy(onesv, degacc.at[pl.ds(row0 + k * _CH, _CH)])
        return 0
      lax.fori_loop(0, zsteps, zcopy16, 0)
      def ostore16(i, _):
        onesv[i, pl.ds(0, 16)] = jnp.ones((16,), jnp.float32)
        return 0
      lax.fori_loop(0, _CH, ostore16, 0)

    plsc.subcore_barrier()

    base = wid * steps
    pltpu.sync_copy(srci.at[pl.ds(base, steps)], srcv)
    pltpu.sync_copy(dsti.at[pl.ds(base, steps)], dstv)

    pltpu.make_async_copy(table.at[srcv.at[0]], gbuf, sem).start()

    def pair(it, _):
      j0 = 2 * it
      pltpu.make_async_copy(table.at[srcv.at[j0 + 1]], gbuf1, sem1).start()
      pltpu.make_async_copy(table.at[srcv.at[j0]], gbuf, sem).wait()
      pltpu.sync_copy(gbuf, acc.at[dstv.at[j0]], add=True)
      if with_deg:
        pltpu.sync_copy(onesv, degacc.at[dstv.at[j0]], add=True)

      @pl.when(it + 1 < steps // 2)
      def _():
        pltpu.make_async_copy(table.at[srcv.at[j0 + 2]], gbuf, sem).start()

      pltpu.make_async_copy(table.at[srcv.at[j0 + 1]], gbuf1, sem1).wait()
      pltpu.sync_copy(gbuf1, acc.at[dstv.at[j0 + 1]], add=True)
      if with_deg:
        pltpu.sync_copy(onesv, degacc.at[dstv.at[j0 + 1]], add=True)
      return 0
    lax.fori_loop(0, steps // 2, pair, 0)

    plsc.subcore_barrier()

    def ocopy(k, _):
      r = row0 + k * _CH
      pltpu.sync_copy(acc.at[pl.ds(r, _CH)], out.at[c, pl.ds(r, _CH)])
      return 0
    lax.fori_loop(0, zsteps, ocopy, 0)
    if with_deg:
      pltpu.sync_copy(degacc.at[pl.ds(row0, rows_per_tile)],
                      degout.at[c, pl.ds(row0, rows_per_tile)])

  return pl.kernel(body, out_type=tuple(out_type), mesh=mesh,
                   scratch_types=tuple(scratch),
                   compiler_params=pltpu.CompilerParams(
                       use_tc_tiling_on_sc=False))


def _tc_layer1(pa, pb, degp, x, wl1t_a, wl1t_b, bl1, wr1t, wl2t, wr2t, n, blk):
  d = x.shape[1]
  dh = d // 2
  h2 = wl2t.shape[1]
  grid = (n // blk,)

  def body(pa_ref, pb_ref, deg_ref, x_ref, wl1a_ref, wl1b_ref, bl1_ref,
           wr1_ref, wl2_ref, wr2_ref, hl_ref, hr_ref):
    agg_a = pa_ref[0] + pa_ref[1]
    agg_b = pb_ref[0] + pb_ref[1]
    dg = deg_ref[0, :, 0:1] + deg_ref[1, :, 0:1]
    rdeg = 1.0 / jnp.maximum(dg, 1.0)
    h = (jnp.dot(agg_a * rdeg, wl1a_ref[...], preferred_element_type=jnp.float32)
         + jnp.dot(agg_b * rdeg, wl1b_ref[...], preferred_element_type=jnp.float32)
         + bl1_ref[...]
         + jnp.dot(x_ref[...], wr1_ref[...], preferred_element_type=jnp.float32))
    h = jnp.maximum(h, 0.0)
    hl_ref[...] = jnp.dot(h, wl2_ref[...], preferred_element_type=jnp.float32)
    hr_ref[...] = jnp.dot(h, wr2_ref[...], preferred_element_type=jnp.float32)

  return pl.pallas_call(
      body,
      grid=grid,
      in_specs=[
          pl.BlockSpec((_NC, blk, dh), lambda i: (0, i, 0)),
          pl.BlockSpec((_NC, blk, dh), lambda i: (0, i, 0)),
          pl.BlockSpec((_NC, blk, _DEGW), lambda i: (0, i, 0)),
          pl.BlockSpec((blk, d), lambda i: (i, 0)),
          pl.BlockSpec((dh, d), lambda i: (0, 0)),
          pl.BlockSpec((dh, d), lambda i: (0, 0)),
          pl.BlockSpec((1, d), lambda i: (0, 0)),
          pl.BlockSpec((d, d), lambda i: (0, 0)),
          pl.BlockSpec((d, h2), lambda i: (0, 0)),
          pl.BlockSpec((d, h2), lambda i: (0, 0)),
      ],
      out_specs=[
          pl.BlockSpec((blk, h2), lambda i: (i, 0)),
          pl.BlockSpec((blk, h2), lambda i: (i, 0)),
      ],
      out_shape=[
          jax.ShapeDtypeStruct((n, h2), jnp.float32),
          jax.ShapeDtypeStruct((n, h2), jnp.float32),
      ],
  )(pa, pb, degp, x, wl1t_a, wl1t_b, bl1, wr1t, wl2t, wr2t)


def _tc_layer2(q, degp, hr, bl2, n, blk):
  c = hr.shape[1]
  grid = (n // blk,)

  def body(q_ref, deg_ref, hr_ref, bl2_ref, o_ref):
    agg = q_ref[0] + q_ref[1]
    dg = deg_ref[0, :, 0:1] + deg_ref[1, :, 0:1]
    rdeg = 1.0 / jnp.maximum(dg, 1.0)
    o = agg * rdeg + bl2_ref[...] + hr_ref[...]
    o_ref[...] = jax.nn.sigmoid(o)

  return pl.pallas_call(
      body,
      grid=grid,
      in_specs=[
          pl.BlockSpec((_NC, blk, c), lambda i: (0, i, 0)),
          pl.BlockSpec((_NC, blk, _DEGW), lambda i: (0, i, 0)),
          pl.BlockSpec((blk, c), lambda i: (i, 0)),
          pl.BlockSpec((1, c), lambda i: (0, 0)),
      ],
      out_specs=pl.BlockSpec((blk, c), lambda i: (i, 0)),
      out_shape=jax.ShapeDtypeStruct((n, c), jnp.float32),
  )(q, degp, hr, bl2)


def kernel(x, edge_index, Wl1, bl1, Wr1, Wl2, bl2, Wr2):
  n, d = x.shape
  dh = d // 2
  h2 = Wl2.shape[0]
  e = edge_index.shape[1]

  egrain = _NC * _NS * _CH * 2
  e_pad = -(-e // egrain) * egrain
  n_acc = -(-(n + 1) // (_NS * _CH)) * (_NS * _CH)

  src = edge_index[0]
  dst = edge_index[1]
  pad = e_pad - e
  src_p = jnp.concatenate([src, jnp.zeros((pad,), jnp.int32)]).reshape(
      e_pad // _CH, _CH)
  dst_p = jnp.concatenate([dst, jnp.full((pad,), n, jnp.int32)]).reshape(
      e_pad // _CH, _CH)

  agg_deg = _make_edge_agg(dh, n_acc, e_pad, with_deg=True)
  agg = _make_edge_agg(dh, n_acc, e_pad, with_deg=False)

  xa = x[:, :dh]
  xb = x[:, dh:]
  pa, degp = agg_deg(xa, src_p, dst_p)
  (pb,) = agg(xb, src_p, dst_p)

  blk = 1000 if n % 1000 == 0 else 8 * (n // 8)
  hl, hr = _tc_layer1(pa, pb, degp, x, Wl1.T[:dh], Wl1.T[dh:],
                      bl1.reshape(1, -1), Wr1.T, Wl2.T, Wr2.T, n, blk)

  (q,) = agg(hl, src_p, dst_p)

  return _tc_layer2(q, degp, hr, bl2.reshape(1, -1), n, blk)


# speedup vs baseline: 10.7507x; 2.3772x over previous
import functools

import jax
import jax.numpy as jnp
from jax import lax
from jax.experimental import pallas as pl
from jax.experimental.pallas import tpu as pltpu
from jax.experimental.pallas import tpu_sc as plsc

_NC = 2
_NS = 16
_CH = 64
_DEGW = 8


def _make_edge_agg(d, n_acc, e_pad):
  steps = e_pad // (_NC * _NS * _CH)
  rows_per_tile = n_acc // _NS
  zrows = rows_per_tile // 2
  zfull = rows_per_tile // _CH
  zrem = rows_per_tile - zfull * _CH
  dpg = d // 16

  mesh = plsc.VectorSubcoreMesh(core_axis_name="c", subcore_axis_name="s",
                                num_cores=_NC, num_subcores=_NS)
  out_type = (
      jax.ShapeDtypeStruct((_NC, n_acc, d), jnp.float32),
      jax.ShapeDtypeStruct((_NC, n_acc, d), jnp.float32),
      jax.ShapeDtypeStruct((_NC, n_acc, _DEGW), jnp.float32),
  )
  scratch = (
      pltpu.VMEM((16,), jnp.int32),
      pltpu.VMEM((steps, _CH), jnp.int32),
      pltpu.VMEM((steps, _CH), jnp.int32),
      pltpu.VMEM((_CH, d), jnp.float32),
      pltpu.VMEM((_CH, d), jnp.float32),
      pltpu.VMEM((_CH, d), jnp.float32),
      pltpu.VMEM((_CH, d), jnp.float32),
      pltpu.VMEM((_CH, d), jnp.float32),
      pltpu.VMEM((_CH, _DEGW), jnp.float32),
      pltpu.VMEM_SHARED((n_acc, d), jnp.float32),
      pltpu.VMEM_SHARED((n_acc, d), jnp.float32),
      pltpu.VMEM_SHARED((n_acc, _DEGW), jnp.float32),
      pltpu.SemaphoreType.DMA,
      pltpu.SemaphoreType.DMA,
      pltpu.SemaphoreType.DMA,
      pltpu.SemaphoreType.DMA,
  )

  def body(ta, tb, srci, dsti, flags, consts, outa, outb, degout,
           flagv, srcv, dstv, zbuf, ga0, ga1, gb0, gb1, onesv,
           acca, accb, degacc, sa0, sa1, sb0, sb1):
    c = lax.axis_index("c")
    s = lax.axis_index("s")
    wid = c * _NS + s
    row0 = s * rows_per_tile

    pltpu.sync_copy(flags, flagv)
    full = flagv[...][0] > 0

    def zstore(i, _):
      zbuf[i // dpg, pl.ds((i % dpg) * 16, 16)] = jnp.zeros((16,), jnp.float32)
      return 0
    lax.fori_loop(0, _CH * dpg, zstore, 0)

    def zcopy(k, _):
      pltpu.sync_copy(zbuf, acca.at[pl.ds(row0 + k * _CH, _CH)])
      return 0
    lax.fori_loop(0, zfull, zcopy, 0)
    if zrem:
      pltpu.sync_copy(zbuf.at[pl.ds(0, zrem)],
                      acca.at[pl.ds(row0 + zfull * _CH, zrem)])

    @pl.when(full)
    def _():
      def zcopyb(k, _):
        pltpu.sync_copy(zbuf, accb.at[pl.ds(row0 + k * _CH, _CH)])
        return 0
      lax.fori_loop(0, zfull, zcopyb, 0)
      if zrem:
        pltpu.sync_copy(zbuf.at[pl.ds(0, zrem)],
                        accb.at[pl.ds(row0 + zfull * _CH, zrem)])
      pltpu.sync_copy(consts.at[0], onesv)
      def zdeg(k, _):
        pltpu.sync_copy(onesv, degacc.at[pl.ds(row0 + k * _CH, _CH)])
        return 0
      lax.fori_loop(0, zfull, zdeg, 0)
      if zrem:
        pltpu.sync_copy(onesv.at[pl.ds(0, zrem)],
                        degacc.at[pl.ds(row0 + zfull * _CH, zrem)])
      pltpu.sync_copy(consts.at[1], onesv)

    plsc.subcore_barrier()

    base = wid * steps
    pltpu.sync_copy(srci.at[pl.ds(base, steps)], srcv)
    pltpu.sync_copy(dsti.at[pl.ds(base, steps)], dstv)

    def start_a(j, buf, sem):
      pltpu.make_async_copy(ta.at[srcv.at[j]], buf, sem).start()

    def start_b(j, buf, sem):
      @pl.when(full)
      def _():
        pltpu.make_async_copy(tb.at[srcv.at[j]], buf, sem).start()

    def drain_scatter(j, bufa, sema, bufb, semb):
      pltpu.make_async_copy(ta.at[srcv.at[j]], bufa, sema).wait()
      pltpu.sync_copy(bufa, acca.at[dstv.at[j]], add=True)
      @pl.when(full)
      def _():
        pltpu.make_async_copy(tb.at[srcv.at[j]], bufb, semb).wait()
        pltpu.sync_copy(bufb, accb.at[dstv.at[j]], add=True)
        pltpu.sync_copy(onesv, degacc.at[dstv.at[j]], add=True)

    start_a(0, ga0, sa0)
    start_b(0, gb0, sb0)

    def pair(it, _):
      j0 = 2 * it
      start_a(j0 + 1, ga1, sa1)
      start_b(j0 + 1, gb1, sb1)
      drain_scatter(j0, ga0, sa0, gb0, sb0)

      @pl.when(it + 1 < steps // 2)
      def _():
        start_a(j0 + 2, ga0, sa0)
        start_b(j0 + 2, gb0, sb0)

      drain_scatter(j0 + 1, ga1, sa1, gb1, sb1)
      return 0
    lax.fori_loop(0, steps // 2, pair, 0)

    plsc.subcore_barrier()

    def ocopy(k, _):
      r = row0 + k * _CH
      pltpu.sync_copy(acca.at[pl.ds(r, _CH)], outa.at[c, pl.ds(r, _CH)])
      return 0
    lax.fori_loop(0, zfull, ocopy, 0)
    if zrem:
      r1 = row0 + zfull * _CH
      pltpu.sync_copy(acca.at[pl.ds(r1, zrem)], outa.at[c, pl.ds(r1, zrem)])

    @pl.when(full)
    def _():
      def ocopyb(k, _):
        r = row0 + k * _CH
        pltpu.sync_copy(accb.at[pl.ds(r, _CH)], outb.at[c, pl.ds(r, _CH)])
        return 0
      lax.fori_loop(0, zfull, ocopyb, 0)
      if zrem:
        r2 = row0 + zfull * _CH
        pltpu.sync_copy(accb.at[pl.ds(r2, zrem)], outb.at[c, pl.ds(r2, zrem)])
      pltpu.sync_copy(degacc.at[pl.ds(row0, rows_per_tile)],
                      degout.at[c, pl.ds(row0, rows_per_tile)])

  return pl.kernel(body, out_type=out_type, mesh=mesh,
                   scratch_types=scratch,
                   compiler_params=pltpu.CompilerParams(
                       use_tc_tiling_on_sc=False))


def _tc_layer1(pa, pb, degp, x, wl1t_a, wl1t_b, bl1, wr1t, wl2t, wr2t, n, blk):
  d = x.shape[1]
  dh = d // 2
  h2 = wl2t.shape[1]
  grid = (n // blk,)

  def body(pa_ref, pb_ref, deg_ref, x_ref, wl1a_ref, wl1b_ref, bl1_ref,
           wr1_ref, wl2_ref, wr2_ref, hl_ref, hr_ref):
    agg_a = pa_ref[0] + pa_ref[1]
    agg_b = pb_ref[0] + pb_ref[1]
    dg = deg_ref[0, :, 0:1] + deg_ref[1, :, 0:1]
    rdeg = 1.0 / jnp.maximum(dg, 1.0)
    h = (jnp.dot(agg_a * rdeg, wl1a_ref[...], preferred_element_type=jnp.float32)
         + jnp.dot(agg_b * rdeg, wl1b_ref[...], preferred_element_type=jnp.float32)
         + bl1_ref[...]
         + jnp.dot(x_ref[...], wr1_ref[...], preferred_element_type=jnp.float32))
    h = jnp.maximum(h, 0.0)
    hl_ref[...] = jnp.dot(h, wl2_ref[...], preferred_element_type=jnp.float32)
    hr_ref[...] = jnp.dot(h, wr2_ref[...], preferred_element_type=jnp.float32)

  return pl.pallas_call(
      body,
      grid=grid,
      in_specs=[
          pl.BlockSpec((_NC, blk, dh), lambda i: (0, i, 0)),
          pl.BlockSpec((_NC, blk, dh), lambda i: (0, i, 0)),
          pl.BlockSpec((_NC, blk, _DEGW), lambda i: (0, i, 0)),
          pl.BlockSpec((blk, d), lambda i: (i, 0)),
          pl.BlockSpec((dh, d), lambda i: (0, 0)),
          pl.BlockSpec((dh, d), lambda i: (0, 0)),
          pl.BlockSpec((1, d), lambda i: (0, 0)),
          pl.BlockSpec((d, d), lambda i: (0, 0)),
          pl.BlockSpec((d, h2), lambda i: (0, 0)),
          pl.BlockSpec((d, h2), lambda i: (0, 0)),
      ],
      out_specs=[
          pl.BlockSpec((blk, h2), lambda i: (i, 0)),
          pl.BlockSpec((blk, h2), lambda i: (i, 0)),
      ],
      out_shape=[
          jax.ShapeDtypeStruct((n, h2), jnp.float32),
          jax.ShapeDtypeStruct((n, h2), jnp.float32),
      ],
  )(pa, pb, degp, x, wl1t_a, wl1t_b, bl1, wr1t, wl2t, wr2t)


def _tc_layer2(q, degp, hr, bl2, n, blk):
  c = hr.shape[1]
  grid = (n // blk,)

  def body(q_ref, deg_ref, hr_ref, bl2_ref, o_ref):
    agg = q_ref[0] + q_ref[1]
    dg = deg_ref[0, :, 0:1] + deg_ref[1, :, 0:1]
    rdeg = 1.0 / jnp.maximum(dg, 1.0)
    o = agg * rdeg + bl2_ref[...] + hr_ref[...]
    o_ref[...] = jax.nn.sigmoid(o)

  return pl.pallas_call(
      body,
      grid=grid,
      in_specs=[
          pl.BlockSpec((_NC, blk, c), lambda i: (0, i, 0)),
          pl.BlockSpec((_NC, blk, _DEGW), lambda i: (0, i, 0)),
          pl.BlockSpec((blk, c), lambda i: (i, 0)),
          pl.BlockSpec((1, c), lambda i: (0, 0)),
      ],
      out_specs=pl.BlockSpec((blk, c), lambda i: (i, 0)),
      out_shape=jax.ShapeDtypeStruct((n, c), jnp.float32),
  )(q, degp, hr, bl2)


def kernel(x, edge_index, Wl1, bl1, Wr1, Wl2, bl2, Wr2):
  n, d = x.shape
  dh = d // 2
  h2 = Wl2.shape[0]
  e = edge_index.shape[1]

  egrain = _NC * _NS * _CH * 2
  e_pad = -(-e // egrain) * egrain
  n_acc = -(-(n + 1) // (2 * _NS)) * (2 * _NS)
  n_dump = n_acc - n

  src = edge_index[0]
  dst = edge_index[1]
  pad = e_pad - e
  pad_iota = jnp.arange(pad, dtype=jnp.int32)
  src_p = jnp.concatenate([src, pad_iota % n]).reshape(e_pad // _CH, _CH)
  dst_p = jnp.concatenate([dst, n + pad_iota % n_dump]).reshape(
      e_pad // _CH, _CH)

  consts = jnp.stack([jnp.zeros((_CH, _DEGW), jnp.float32),
                      jnp.ones((_CH, _DEGW), jnp.float32)])

  agg = _make_edge_agg(dh, n_acc, e_pad)

  flags_on = jnp.ones((16,), jnp.int32)
  flags_off = jnp.zeros((16,), jnp.int32)

  pa, pb, degp = agg(x[:, :dh], x[:, dh:], src_p, dst_p, flags_on, consts)

  blk = 1000 if n % 1000 == 0 else 8 * (n // 8)
  hl, hr = _tc_layer1(pa, pb, degp, x, Wl1.T[:dh], Wl1.T[dh:],
                      bl1.reshape(1, -1), Wr1.T, Wl2.T, Wr2.T, n, blk)

  q, _, _ = agg(hl, hl, src_p, dst_p, flags_off, consts)

  return _tc_layer2(q, degp, hr, bl2.reshape(1, -1), n, blk)


# speedup vs baseline: 10.9321x; 1.0169x over previous
import functools

import jax
import jax.numpy as jnp
from jax import lax
from jax.experimental import pallas as pl
from jax.experimental.pallas import tpu as pltpu
from jax.experimental.pallas import tpu_sc as plsc

_NC = 2
_NS = 16
_CH = 64
_DEGW = 8


def _make_edge_agg(d, n_acc, e_pad):
  steps = e_pad // (_NC * _NS * _CH)
  rows_per_tile = n_acc // _NS
  zrows = rows_per_tile // 2
  zfull = rows_per_tile // _CH
  zrem = rows_per_tile - zfull * _CH
  dpg = d // 16

  mesh = plsc.VectorSubcoreMesh(core_axis_name="c", subcore_axis_name="s",
                                num_cores=_NC, num_subcores=_NS)
  out_type = (
      jax.ShapeDtypeStruct((_NC, n_acc, d), jnp.float32),
      jax.ShapeDtypeStruct((_NC, n_acc, d), jnp.float32),
      jax.ShapeDtypeStruct((_NC, n_acc, _DEGW), jnp.float32),
  )
  scratch = (
      pltpu.VMEM((16,), jnp.int32),
      pltpu.VMEM((steps, _CH), jnp.int32),
      pltpu.VMEM((steps, _CH), jnp.int32),
      pltpu.VMEM((_CH, d), jnp.float32),
      pltpu.VMEM((_CH, d), jnp.float32),
      pltpu.VMEM((_CH, d), jnp.float32),
      pltpu.VMEM((_CH, d), jnp.float32),
      pltpu.VMEM((_CH, d), jnp.float32),
      pltpu.VMEM((_CH, _DEGW), jnp.float32),
      pltpu.VMEM_SHARED((n_acc, d), jnp.float32),
      pltpu.VMEM_SHARED((n_acc, d), jnp.float32),
      pltpu.VMEM_SHARED((n_acc, _DEGW), jnp.float32),
      pltpu.SemaphoreType.DMA,
      pltpu.SemaphoreType.DMA,
      pltpu.SemaphoreType.DMA,
      pltpu.SemaphoreType.DMA,
  )

  def body(ta, tb, srci, dsti, flags, consts, outa, outb, degout,
           flagv, srcv, dstv, zbuf, ga0, ga1, gb0, gb1, onesv,
           acca, accb, degacc, sa0, sa1, sb0, sb1):
    c = lax.axis_index("c")
    s = lax.axis_index("s")
    wid = c * _NS + s
    row0 = s * rows_per_tile

    pltpu.sync_copy(flags, flagv)
    full = flagv[...][0] > 0

    def zstore(i, _):
      zbuf[i // dpg, pl.ds((i % dpg) * 16, 16)] = jnp.zeros((16,), jnp.float32)
      return 0
    lax.fori_loop(0, _CH * dpg, zstore, 0)

    def zcopy(k, _):
      pltpu.sync_copy(zbuf, acca.at[pl.ds(row0 + k * _CH, _CH)])
      return 0
    lax.fori_loop(0, zfull, zcopy, 0)
    if zrem:
      pltpu.sync_copy(zbuf.at[pl.ds(0, zrem)],
                      acca.at[pl.ds(row0 + zfull * _CH, zrem)])

    @pl.when(full)
    def _():
      def zcopyb(k, _):
        pltpu.sync_copy(zbuf, accb.at[pl.ds(row0 + k * _CH, _CH)])
        return 0
      lax.fori_loop(0, zfull, zcopyb, 0)
      if zrem:
        pltpu.sync_copy(zbuf.at[pl.ds(0, zrem)],
                        accb.at[pl.ds(row0 + zfull * _CH, zrem)])
      pltpu.sync_copy(consts.at[0], onesv)
      def zdeg(k, _):
        pltpu.sync_copy(onesv, degacc.at[pl.ds(row0 + k * _CH, _CH)])
        return 0
      lax.fori_loop(0, zfull, zdeg, 0)
      if zrem:
        pltpu.sync_copy(onesv.at[pl.ds(0, zrem)],
                        degacc.at[pl.ds(row0 + zfull * _CH, zrem)])
      pltpu.sync_copy(consts.at[1], onesv)

    plsc.subcore_barrier()

    base = wid * steps
    pltpu.sync_copy(srci.at[pl.ds(base, steps)], srcv)
    pltpu.sync_copy(dsti.at[pl.ds(base, steps)], dstv)

    def start_a(j, buf, sem):
      pltpu.make_async_copy(ta.at[srcv.at[j]], buf, sem).start()

    def start_b(j, buf, sem):
      @pl.when(full)
      def _():
        pltpu.make_async_copy(tb.at[srcv.at[j]], buf, sem).start()

    def drain_scatter(j, bufa, sema, bufb, semb):
      pltpu.make_async_copy(ta.at[srcv.at[j]], bufa, sema).wait()
      pltpu.sync_copy(bufa, acca.at[dstv.at[j]], add=True)
      @pl.when(full)
      def _():
        pltpu.make_async_copy(tb.at[srcv.at[j]], bufb, semb).wait()
        pltpu.sync_copy(bufb, accb.at[dstv.at[j]], add=True)
        pltpu.sync_copy(onesv, degacc.at[dstv.at[j]], add=True)

    start_a(0, ga0, sa0)
    start_b(0, gb0, sb0)

    def pair(it, _):
      j0 = 2 * it
      start_a(j0 + 1, ga1, sa1)
      start_b(j0 + 1, gb1, sb1)
      drain_scatter(j0, ga0, sa0, gb0, sb0)

      @pl.when(it + 1 < steps // 2)
      def _():
        start_a(j0 + 2, ga0, sa0)
        start_b(j0 + 2, gb0, sb0)

      drain_scatter(j0 + 1, ga1, sa1, gb1, sb1)
      return 0
    lax.fori_loop(0, steps // 2, pair, 0)

    plsc.subcore_barrier()

    def ocopy(k, _):
      r = row0 + k * _CH
      pltpu.sync_copy(acca.at[pl.ds(r, _CH)], outa.at[c, pl.ds(r, _CH)])
      return 0
    lax.fori_loop(0, zfull, ocopy, 0)
    if zrem:
      r1 = row0 + zfull * _CH
      pltpu.sync_copy(acca.at[pl.ds(r1, zrem)], outa.at[c, pl.ds(r1, zrem)])

    @pl.when(full)
    def _():
      def ocopyb(k, _):
        r = row0 + k * _CH
        pltpu.sync_copy(accb.at[pl.ds(r, _CH)], outb.at[c, pl.ds(r, _CH)])
        return 0
      lax.fori_loop(0, zfull, ocopyb, 0)
      if zrem:
        r2 = row0 + zfull * _CH
        pltpu.sync_copy(accb.at[pl.ds(r2, zrem)], outb.at[c, pl.ds(r2, zrem)])
      pltpu.sync_copy(degacc.at[pl.ds(row0, rows_per_tile)],
                      degout.at[c, pl.ds(row0, rows_per_tile)])

  return pl.kernel(body, out_type=out_type, mesh=mesh,
                   scratch_types=scratch,
                   compiler_params=pltpu.CompilerParams(
                       use_tc_tiling_on_sc=False))


def _tc_layer1(pa, pb, degp, x, wl1t_a, wl1t_b, bl1, wr1t, wl2t, wr2t, n, blk):
  d = x.shape[1]
  dh = d // 2
  h2 = wl2t.shape[1]
  grid = (n // blk,)

  def body(pa_ref, pb_ref, deg_ref, x_ref, wl1a_ref, wl1b_ref, bl1_ref,
           wr1_ref, wl2_ref, wr2_ref, hl_ref, hr_ref):
    agg_a = pa_ref[0] + pa_ref[1]
    agg_b = pb_ref[0] + pb_ref[1]
    dg = deg_ref[0, :, 0:1] + deg_ref[1, :, 0:1]
    rdeg = 1.0 / jnp.maximum(dg, 1.0)
    h = (jnp.dot(agg_a * rdeg, wl1a_ref[...], preferred_element_type=jnp.float32)
         + jnp.dot(agg_b * rdeg, wl1b_ref[...], preferred_element_type=jnp.float32)
         + bl1_ref[...]
         + jnp.dot(x_ref[...], wr1_ref[...], preferred_element_type=jnp.float32))
    h = jnp.maximum(h, 0.0)
    hl_ref[...] = jnp.dot(h, wl2_ref[...], preferred_element_type=jnp.float32)
    hr_ref[...] = jnp.dot(h, wr2_ref[...], preferred_element_type=jnp.float32)

  return pl.pallas_call(
      body,
      grid=grid,
      in_specs=[
          pl.BlockSpec((_NC, blk, dh), lambda i: (0, i, 0)),
          pl.BlockSpec((_NC, blk, dh), lambda i: (0, i, 0)),
          pl.BlockSpec((_NC, blk, _DEGW), lambda i: (0, i, 0)),
          pl.BlockSpec((blk, d), lambda i: (i, 0)),
          pl.BlockSpec((dh, d), lambda i: (0, 0)),
          pl.BlockSpec((dh, d), lambda i: (0, 0)),
          pl.BlockSpec((1, d), lambda i: (0, 0)),
          pl.BlockSpec((d, d), lambda i: (0, 0)),
          pl.BlockSpec((d, h2), lambda i: (0, 0)),
          pl.BlockSpec((d, h2), lambda i: (0, 0)),
      ],
      out_specs=[
          pl.BlockSpec((blk, h2), lambda i: (i, 0)),
          pl.BlockSpec((blk, h2), lambda i: (i, 0)),
      ],
      out_shape=[
          jax.ShapeDtypeStruct((n, h2), jnp.float32),
          jax.ShapeDtypeStruct((n, h2), jnp.float32),
      ],
  )(pa, pb, degp, x, wl1t_a, wl1t_b, bl1, wr1t, wl2t, wr2t)


def _tc_layer2(q, degp, hr, bl2, n, blk):
  c = hr.shape[1]
  grid = (n // blk,)

  def body(q_ref, deg_ref, hr_ref, bl2_ref, o_ref):
    agg = q_ref[0] + q_ref[1]
    dg = deg_ref[0, :, 0:1] + deg_ref[1, :, 0:1]
    rdeg = 1.0 / jnp.maximum(dg, 1.0)
    o = agg * rdeg + bl2_ref[...] + hr_ref[...]
    o_ref[...] = jax.nn.sigmoid(o)

  return pl.pallas_call(
      body,
      grid=grid,
      in_specs=[
          pl.BlockSpec((_NC, blk, c), lambda i: (0, i, 0)),
          pl.BlockSpec((_NC, blk, _DEGW), lambda i: (0, i, 0)),
          pl.BlockSpec((blk, c), lambda i: (i, 0)),
          pl.BlockSpec((1, c), lambda i: (0, 0)),
      ],
      out_specs=pl.BlockSpec((blk, c), lambda i: (i, 0)),
      out_shape=jax.ShapeDtypeStruct((n, c), jnp.float32),
  )(q, degp, hr, bl2)


def kernel(x, edge_index, Wl1, bl1, Wr1, Wl2, bl2, Wr2):
  n, d = x.shape
  dh = d // 2
  h2 = Wl2.shape[0]
  e = edge_index.shape[1]

  egrain = _NC * _NS * _CH * 2
  e_pad = -(-e // egrain) * egrain
  n_acc = -(-(n + 1) // (2 * _NS)) * (2 * _NS)
  n_dump = n_acc - n

  src = edge_index[0]
  dst = edge_index[1]
  pad = e_pad - e
  pad_iota = jnp.arange(pad, dtype=jnp.int32)
  src_p = jnp.concatenate([src, pad_iota % n]).reshape(e_pad // _CH, _CH)
  dst_p = jnp.concatenate([dst, n + pad_iota % n_dump]).reshape(
      e_pad // _CH, _CH)

  consts = jnp.stack([jnp.zeros((_CH, _DEGW), jnp.float32),
                      jnp.ones((_CH, _DEGW), jnp.float32)])

  agg = _make_edge_agg(dh, n_acc, e_pad)

  flags_on = jnp.ones((16,), jnp.int32)
  flags_off = jnp.zeros((16,), jnp.int32)

  pa, pb, degp = agg(x[:, :dh], x[:, dh:], src_p, dst_p, flags_on, consts)

  blk = 2000 if n % 2000 == 0 else 8 * (n // 8)
  hl, hr = _tc_layer1(pa, pb, degp, x, Wl1.T[:dh], Wl1.T[dh:],
                      bl1.reshape(1, -1), Wr1.T, Wl2.T, Wr2.T, n, blk)

  q, _, _ = agg(hl, hl, src_p, dst_p, flags_off, consts)

  return _tc_layer2(q, degp, hr, bl2.reshape(1, -1), n, blk)
